# baseline (speedup 1.0000x reference)
import jax
import jax.numpy as jnp
from jax.experimental import pallas as pl
from jax.experimental.pallas import tpu as pltpu

N = 8192
D = 1024
E = 8
K = 2
LOSS_COEF = 0.01

_TILE = 2048


def _moe_kernel(x_ref, wg_ref, bg_ref, w_ref, b_ref,
                gates_ref, psum_ref, o_ref, g_scr):
    e = pl.program_id(1)

    @pl.when(e == 0)
    def _():
        logits = jnp.dot(x_ref[...], wg_ref[...],
                         preferred_element_type=jnp.float32) + bg_ref[...]
        iota = jax.lax.broadcasted_iota(jnp.int32, logits.shape, 1)
        v1 = jnp.max(logits, axis=1, keepdims=True)
        i1 = jnp.argmax(logits, axis=1).astype(jnp.int32)[:, None]
        masked = jnp.where(iota == i1, -jnp.inf, logits)
        v2 = jnp.max(masked, axis=1, keepdims=True)
        i2 = jnp.argmax(masked, axis=1).astype(jnp.int32)[:, None]
        e2 = jnp.exp(v2 - v1)
        g1 = 1.0 / (1.0 + e2)
        g2 = e2 * g1
        gates = (jnp.where(iota == i1, g1, 0.0)
                 + jnp.where(iota == i2, g2, 0.0))
        gates_ref[...] = gates
        g_scr[...] = gates
        psum_ref[0, 0, :] = jnp.sum(gates, axis=0)

    xw = jnp.dot(x_ref[...], w_ref[0],
                 preferred_element_type=jnp.float32) + b_ref[0]
    iota = jax.lax.broadcasted_iota(jnp.int32, g_scr.shape, 1)
    gcol = jnp.sum(jnp.where(iota == e, g_scr[...], 0.0), axis=1,
                   keepdims=True)

    @pl.when(e == 0)
    def _():
        o_ref[...] = gcol * xw

    @pl.when(e != 0)
    def _():
        o_ref[...] += gcol * xw


@jax.jit
def kernel(x, w_gate_W, w_gate_b, expert_W, expert_b):
    n_t = N // _TILE
    gates, psums, y = pl.pallas_call(
        _moe_kernel,
        grid=(n_t, E),
        in_specs=[
            pl.BlockSpec((_TILE, D), lambda i, e: (i, 0)),
            pl.BlockSpec((D, E), lambda i, e: (0, 0)),
            pl.BlockSpec((1, E), lambda i, e: (0, 0)),
            pl.BlockSpec((1, D, D), lambda i, e: (e, 0, 0)),
            pl.BlockSpec((1, 1, D), lambda i, e: (e, 0, 0)),
        ],
        out_specs=[
            pl.BlockSpec((_TILE, E), lambda i, e: (i, 0)),
            pl.BlockSpec((1, 1, E), lambda i, e: (i, 0, 0)),
            pl.BlockSpec((_TILE, D), lambda i, e: (i, 0)),
        ],
        out_shape=[
            jax.ShapeDtypeStruct((N, E), jnp.float32),
            jax.ShapeDtypeStruct((n_t, 1, E), jnp.float32),
            jax.ShapeDtypeStruct((N, D), jnp.float32),
        ],
        scratch_shapes=[pltpu.VMEM((_TILE, E), jnp.float32)],
        compiler_params=pltpu.CompilerParams(
            dimension_semantics=("parallel", "arbitrary")),
    )(x, w_gate_W, w_gate_b.reshape(1, E), expert_W,
      expert_b.reshape(E, 1, D))

    importance = jnp.sum(psums[:, 0, :], axis=0) / N
    loss = (jnp.std(importance, ddof=1) / jnp.mean(importance)) * LOSS_COEF
    return (y, loss, gates)


# speedup vs baseline: 1.1175x; 1.1175x over previous
import jax
import jax.numpy as jnp
from jax.experimental import pallas as pl
from jax.experimental.pallas import tpu as pltpu

N = 8192
D = 1024
E = 8
K = 2
LOSS_COEF = 0.01

_TILE = 2048


def _moe_kernel(x_ref, wg_ref, bg_ref, w_ref, b_ref,
                gates_ref, psum_ref, o_ref, g_scr):
    e = pl.program_id(1)

    @pl.when(e == 0)
    def _():
        logits = jnp.dot(x_ref[...], wg_ref[...],
                         preferred_element_type=jnp.float32) + bg_ref[...]
        iota = jax.lax.broadcasted_iota(jnp.int32, logits.shape, 1)
        v1 = jnp.max(logits, axis=1, keepdims=True)
        i1 = jnp.argmax(logits, axis=1).astype(jnp.int32)[:, None]
        masked = jnp.where(iota == i1, -jnp.inf, logits)
        v2 = jnp.max(masked, axis=1, keepdims=True)
        i2 = jnp.argmax(masked, axis=1).astype(jnp.int32)[:, None]
        e2 = jnp.exp(v2 - v1)
        g1 = 1.0 / (1.0 + e2)
        g2 = e2 * g1
        gates = (jnp.where(iota == i1, g1, 0.0)
                 + jnp.where(iota == i2, g2, 0.0))
        gates_ref[...] = gates
        g_scr[...] = gates
        psum_ref[0, 0, :] = jnp.sum(gates, axis=0)
        o_ref[...] = jnp.zeros_like(o_ref)

    xw = jnp.dot(x_ref[...], w_ref[0],
                 preferred_element_type=jnp.float32) + b_ref[0]
    iota = jax.lax.broadcasted_iota(jnp.int32, g_scr.shape, 1)
    gcol = jnp.sum(jnp.where(iota == e, g_scr[...], 0.0), axis=1,
                   keepdims=True)
    o_ref[...] += gcol * xw


@jax.jit
def kernel(x, w_gate_W, w_gate_b, expert_W, expert_b):
    n_t = N // _TILE
    gates, psums, y = pl.pallas_call(
        _moe_kernel,
        grid=(n_t, E),
        in_specs=[
            pl.BlockSpec((_TILE, D), lambda i, e: (i, 0)),
            pl.BlockSpec((D, E), lambda i, e: (0, 0)),
            pl.BlockSpec((1, E), lambda i, e: (0, 0)),
            pl.BlockSpec((1, D, D), lambda i, e: (e, 0, 0)),
            pl.BlockSpec((1, 1, D), lambda i, e: (e, 0, 0)),
        ],
        out_specs=[
            pl.BlockSpec((_TILE, E), lambda i, e: (i, 0)),
            pl.BlockSpec((1, 1, E), lambda i, e: (i, 0, 0)),
            pl.BlockSpec((_TILE, D), lambda i, e: (i, 0)),
        ],
        out_shape=[
            jax.ShapeDtypeStruct((N, E), jnp.float32),
            jax.ShapeDtypeStruct((n_t, 1, E), jnp.float32),
            jax.ShapeDtypeStruct((N, D), jnp.float32),
        ],
        scratch_shapes=[pltpu.VMEM((_TILE, E), jnp.float32)],
        compiler_params=pltpu.CompilerParams(
            dimension_semantics=("parallel", "arbitrary")),
    )(x, w_gate_W, w_gate_b.reshape(1, E), expert_W,
      expert_b.reshape(E, 1, D))

    importance = jnp.sum(psums[:, 0, :], axis=0) / N
    loss = (jnp.std(importance, ddof=1) / jnp.mean(importance)) * LOSS_COEF
    return (y, loss, gates)
